# Initial kernel scaffold; baseline (speedup 1.0000x reference)
import jax
import jax.numpy as jnp
from jax.experimental import pallas as pl


def kernel(x, seg, tok_table, seg_table):
    raise NotImplementedError("write your pallas kernel here")


# speedup vs baseline: 2.2351x; 2.2351x over previous
import functools

import jax
import jax.numpy as jnp
from jax import lax
from jax.experimental import pallas as pl
from jax.experimental.pallas import tpu as pltpu
from jax.experimental.pallas import tpu_sc as plsc

B, S = 4, 4096
DIM = 128
N_TOTAL = B * S

_info = plsc.get_sparse_core_info()
_NC, _NS = _info.num_cores, _info.num_subcores
_NW = _NC * _NS
_PER_W = N_TOTAL // _NW


def _emb_kernel(x_hbm, seg_hbm, tok_hbm, segtab_hbm, out_hbm,
                tok_idx_v, seg_idx_v, rows_v, sem):
    wid = lax.axis_index("s") * _NC + lax.axis_index("c")
    base = wid * _PER_W
    pltpu.sync_copy(x_hbm.at[pl.ds(base, _PER_W)], tok_idx_v)
    pltpu.sync_copy(seg_hbm.at[pl.ds(base, _PER_W)], seg_idx_v)
    pltpu.async_copy(tok_hbm.at[tok_idx_v], rows_v, sem).wait()
    pltpu.async_copy(segtab_hbm.at[seg_idx_v], rows_v, sem, add=True).wait()
    pltpu.sync_copy(rows_v, out_hbm.at[pl.ds(base, _PER_W)])


@jax.jit
def _embeddings(x_flat, seg_flat, tok_table, seg_table):
    mesh = plsc.VectorSubcoreMesh(core_axis_name="c", subcore_axis_name="s")
    return pl.kernel(
        _emb_kernel,
        out_type=jax.ShapeDtypeStruct((N_TOTAL, DIM), jnp.float32),
        mesh=mesh,
        scratch_types=[
            pltpu.VMEM((_PER_W,), jnp.int32),
            pltpu.VMEM((_PER_W,), jnp.int32),
            pltpu.VMEM((_PER_W, DIM), jnp.float32),
            pltpu.SemaphoreType.DMA,
        ],
    )(x_flat, seg_flat, tok_table, seg_table)


def kernel(x, seg, tok_table, seg_table):
    out = _embeddings(x.reshape(-1), seg.reshape(-1), tok_table, seg_table)
    return out.reshape(B, S, DIM)
